# baseline (speedup 1.0000x reference)
import functools

import jax
import jax.numpy as jnp
from jax import lax
from jax.experimental import pallas as pl
from jax.experimental.pallas import tpu as pltpu
from jax.experimental.pallas import tpu_sc as plsc

_EPS = 1e-6


def _make_sc_kernel(B, V, D, n_cores, n_subcores):
    nw = n_cores * n_subcores
    b_per_w = B // nw
    n_grp = b_per_w // 16
    mesh = plsc.VectorSubcoreMesh(core_axis_name="c", subcore_axis_name="s")

    @functools.partial(
        pl.kernel,
        mesh=mesh,
        compiler_params=pltpu.CompilerParams(needs_layout_passes=False),
        out_type=[
            jax.ShapeDtypeStruct((D * B,), jnp.float32),
            jax.ShapeDtypeStruct((D * B,), jnp.float32),
        ],
        scratch_types=[
            pltpu.VMEM((b_per_w,), jnp.int32),
            pltpu.VMEM((D * b_per_w,), jnp.int32),
            pltpu.VMEM((D * b_per_w,), jnp.float32),
            pltpu.VMEM((D * b_per_w,), jnp.float32),
            pltpu.VMEM((D * b_per_w,), jnp.float32),
            pltpu.SemaphoreType.DMA,
            pltpu.SemaphoreType.DMA,
        ],
    )
    def sc_kernel(idx_hbm, tflat_hbm, pz_hbm, z_hbm,
                  idx_v, eidx_v, x_v, pz_v, z_v, gsem, osem):
        wid = lax.axis_index("s") * n_cores + lax.axis_index("c")
        base = wid * b_per_w
        pltpu.sync_copy(idx_hbm.at[pl.ds(base, b_per_w)], idx_v)

        def mk_eidx(j, carry):
            v = idx_v[pl.ds(j * 16, 16)]
            for f in range(D):
                eidx_v[pl.ds(f * b_per_w + j * 16, 16)] = v + (f * V)
            return carry

        lax.fori_loop(0, n_grp, mk_eidx, 0)
        pltpu.async_copy(tflat_hbm.at[eidx_v], x_v, gsem).wait()

        def body(i, carry):
            x = x_v[pl.ds(i * 16, 16)]
            p = 1.0 / (1.0 + jnp.exp(-x))
            p = p * (1.0 - 2.0 * _EPS) + _EPS
            pz_v[pl.ds(i * 16, 16)] = p
            z_v[pl.ds(i * 16, 16)] = jnp.where(p > 0.5, 1.0, 0.0)
            return carry

        lax.fori_loop(0, D * n_grp, body, 0)

        copies = []
        for f in range(D):
            src_pz = pz_v.at[pl.ds(f * b_per_w, b_per_w)]
            dst_pz = pz_hbm.at[pl.ds(f * B + base, b_per_w)]
            copies.append(pltpu.async_copy(src_pz, dst_pz, osem))
            src_z = z_v.at[pl.ds(f * b_per_w, b_per_w)]
            dst_z = z_hbm.at[pl.ds(f * B + base, b_per_w)]
            copies.append(pltpu.async_copy(src_z, dst_z, osem))
        for c in copies:
            c.wait()

    return sc_kernel


def kernel(inputs, embedding):
    B = inputs.shape[0]
    V, D = embedding.shape
    info = plsc.get_sparse_core_info()
    idx = inputs.reshape(-1).astype(jnp.int32)
    tflat = embedding.T.reshape(-1)
    sc = _make_sc_kernel(B, V, D, info.num_cores, info.num_subcores)
    pz_t, z_t = sc(idx, tflat)
    pz = pz_t.reshape(D, B).T
    z = z_t.reshape(D, B).T.astype(jnp.bool_)
    return pz, z


# speedup vs baseline: 2.6682x; 2.6682x over previous
import functools

import jax
import jax.numpy as jnp
from jax import lax
from jax.experimental import pallas as pl
from jax.experimental.pallas import tpu as pltpu
from jax.experimental.pallas import tpu_sc as plsc

_EPS = 1e-6


def _make_sc_kernel(B, V, D, n_cores, n_subcores):
    nw = n_cores * n_subcores
    b_per_w = B // nw
    mesh = plsc.VectorSubcoreMesh(core_axis_name="c", subcore_axis_name="s")

    @functools.partial(
        pl.kernel,
        mesh=mesh,
        compiler_params=pltpu.CompilerParams(use_tc_tiling_on_sc=False),
        out_type=[
            jax.ShapeDtypeStruct((B, D), jnp.float32),
            jax.ShapeDtypeStruct((B, D), jnp.float32),
        ],
        scratch_types=[
            pltpu.VMEM((b_per_w,), jnp.int32),
            pltpu.VMEM((b_per_w, D), jnp.float32),
            pltpu.VMEM((b_per_w, D), jnp.float32),
            pltpu.VMEM((b_per_w, D), jnp.float32),
            pltpu.SemaphoreType.DMA,
        ],
    )
    def sc_kernel(idx_hbm, table_hbm, pz_hbm, z_hbm, idx_v, rows_v, pz_v, z_v, sem):
        wid = lax.axis_index("s") * n_cores + lax.axis_index("c")
        base = wid * b_per_w
        pltpu.sync_copy(idx_hbm.at[pl.ds(base, b_per_w)], idx_v)
        pltpu.async_copy(table_hbm.at[idx_v], rows_v, sem).wait()

        def body(i, carry):
            x = rows_v[i, :]
            p = 1.0 / (1.0 + jnp.exp(-x))
            p = p * (1.0 - 2.0 * _EPS) + _EPS
            pz_v[i, :] = p
            z_v[i, :] = jnp.where(p > 0.5, 1.0, 0.0)
            return carry

        lax.fori_loop(0, b_per_w, body, 0)

        pltpu.sync_copy(pz_v, pz_hbm.at[pl.ds(base, b_per_w)])
        pltpu.sync_copy(z_v, z_hbm.at[pl.ds(base, b_per_w)])

    return sc_kernel


def kernel(inputs, embedding):
    B = inputs.shape[0]
    V, D = embedding.shape
    info = plsc.get_sparse_core_info()
    idx = inputs.reshape(-1).astype(jnp.int32)
    sc = _make_sc_kernel(B, V, D, info.num_cores, info.num_subcores)
    pz, z_f = sc(idx, embedding)
    return pz, z_f.astype(jnp.bool_)
